# baseline (speedup 1.0000x reference)
import functools

import jax
import jax.numpy as jnp
from jax import lax
from jax.experimental import pallas as pl
from jax.experimental.pallas import tpu as pltpu
from jax.experimental.pallas import tpu_sc as plsc

D_MODEL = 128
UNIT = 256
NBUF = 3
WLAG = 1


@functools.cache
def _make_gather(n_rows):
    info = plsc.get_sparse_core_info()
    nc, ns = info.num_cores, info.num_subcores
    nw = nc * ns
    rows_per_w = n_rows // nw
    m = rows_per_w // UNIT
    mesh = plsc.VectorSubcoreMesh(core_axis_name="c", subcore_axis_name="s")

    def body(x_hbm, w_hbm, out_hbm, idx_v, rows_v, gsem, wsem):
        wid = lax.axis_index("s") * nc + lax.axis_index("c")
        base = wid * rows_per_w
        pltpu.sync_copy(x_hbm.at[pl.ds(base, rows_per_w)], idx_v)

        def start_gather(j, s):
            pltpu.async_copy(
                w_hbm.at[idx_v.at[pl.ds(j * UNIT, UNIT)]], rows_v.at[s], gsem
            )

        def start_write(j, s):
            pltpu.async_copy(
                rows_v.at[s], out_hbm.at[pl.ds(base + j * UNIT, UNIT)], wsem
            )

        def wait_gather(s):
            pltpu.make_async_copy(
                w_hbm.at[idx_v.at[pl.ds(0, UNIT)]], rows_v.at[s], gsem
            ).wait()

        def wait_write(s):
            pltpu.make_async_copy(
                rows_v.at[s], out_hbm.at[pl.ds(0, UNIT)], wsem
            ).wait()

        for s in range(NBUF):
            start_gather(s, s)

        wait_gather(0)
        start_write(0, 0)

        def steady(j, carry):
            s = lax.rem(j, NBUF)
            wait_gather(s)
            start_write(j, s)
            jd = j - WLAG
            sd = lax.rem(jd, NBUF)
            wait_write(sd)
            start_gather(jd + NBUF, sd)
            return carry

        tail = m - NBUF + WLAG
        lax.fori_loop(1, tail, steady, 0)

        for j in range(tail, m):
            s = j % NBUF
            wait_gather(s)
            start_write(j, s)
            wait_write((j - WLAG) % NBUF)
        for r in range(WLAG):
            wait_write((m - WLAG + r) % NBUF)

    return pl.kernel(
        body,
        out_type=jax.ShapeDtypeStruct((n_rows, D_MODEL), jnp.float32),
        mesh=mesh,
        scratch_types=[
            pltpu.VMEM((rows_per_w,), jnp.int32),
            pltpu.VMEM((NBUF, UNIT, D_MODEL), jnp.float32),
            pltpu.SemaphoreType.DMA,
            pltpu.SemaphoreType.DMA,
        ],
    )


def kernel(x, W):
    b, t = x.shape
    n = b * t
    x_flat = x.reshape(n).astype(jnp.int32)
    out = _make_gather(n)(x_flat, W)
    return out.reshape(b, t, D_MODEL)


# speedup vs baseline: 1.0507x; 1.0507x over previous
import functools

import jax
import jax.numpy as jnp
from jax import lax
from jax.experimental import pallas as pl
from jax.experimental.pallas import tpu as pltpu
from jax.experimental.pallas import tpu_sc as plsc

D_MODEL = 128
UNIT = 128
NB = 3


@functools.cache
def _make_gather(n_rows):
    info = plsc.get_sparse_core_info()
    nc, ns = info.num_cores, info.num_subcores
    nw = nc * ns
    rows_per_w = n_rows // nw
    m = rows_per_w // UNIT
    mesh = plsc.VectorSubcoreMesh(core_axis_name="c", subcore_axis_name="s")

    def body(x_hbm, w_hbm, out_hbm, idx_v, tbuf, sbuf, gsem, xsem, wsem):
        sid = lax.axis_index("s")
        wid = sid * nc + lax.axis_index("c")
        base = wid * rows_per_w
        pltpu.sync_copy(x_hbm.at[pl.ds(base, rows_per_w)], idx_v)

        def start_g(j, t):
            pltpu.async_copy(
                w_hbm.at[idx_v.at[pl.ds(j * UNIT, UNIT)]], tbuf.at[t], gsem
            )

        def wait_g(t):
            pltpu.make_async_copy(
                w_hbm.at[idx_v.at[pl.ds(0, UNIT)]], tbuf.at[t], gsem
            ).wait()

        def start_x(t, s):
            pltpu.async_copy(tbuf.at[t], sbuf.at[sid, s], xsem)

        def wait_x(t, s):
            pltpu.make_async_copy(tbuf.at[t], sbuf.at[sid, s], xsem).wait()

        def start_w(j, s):
            pltpu.async_copy(
                sbuf.at[sid, s], out_hbm.at[pl.ds(base + j * UNIT, UNIT)], wsem
            )

        def wait_w(s):
            pltpu.make_async_copy(
                sbuf.at[sid, s], out_hbm.at[pl.ds(0, UNIT)], wsem
            ).wait()

        for t in range(NB):
            start_g(t, t)
        wait_g(0)
        start_x(0, 0)
        wait_g(1)
        start_x(1, 1)
        wait_x(0, 0)
        start_w(0, 0)
        start_g(NB, 0)

        def steady(j, carry):
            t = lax.rem(j, NB)
            wait_g(t)
            start_x(t, t)
            tp = lax.rem(j - 1, NB)
            wait_x(tp, tp)
            start_w(j - 1, tp)
            start_g(j + 2, tp)
            wait_w(lax.rem(j - 2, NB))
            return carry

        lax.fori_loop(2, m - 2, steady, 0)

        for j in (m - 2, m - 1):
            t = j % NB
            wait_g(t)
            start_x(t, t)
            tp = (j - 1) % NB
            wait_x(tp, tp)
            start_w(j - 1, tp)
            wait_w((j - 2) % NB)
        tl = (m - 1) % NB
        wait_x(tl, tl)
        start_w(m - 1, tl)
        wait_w((m - 2) % NB)
        wait_w((m - 1) % NB)

    return pl.kernel(
        body,
        out_type=jax.ShapeDtypeStruct((n_rows, D_MODEL), jnp.float32),
        mesh=mesh,
        scratch_types=[
            pltpu.VMEM((rows_per_w,), jnp.int32),
            pltpu.VMEM((NB, UNIT, D_MODEL), jnp.float32),
            pltpu.VMEM_SHARED((ns, NB, UNIT, D_MODEL), jnp.float32),
            pltpu.SemaphoreType.DMA,
            pltpu.SemaphoreType.DMA,
            pltpu.SemaphoreType.DMA,
        ],
    )


def kernel(x, W):
    b, t = x.shape
    n = b * t
    x_flat = x.reshape(n).astype(jnp.int32)
    out = _make_gather(n)(x_flat, W)
    return out.reshape(b, t, D_MODEL)
